# baseline (speedup 1.0000x reference)
import functools

import jax
import jax.numpy as jnp
from jax import lax
from jax.experimental import pallas as pl
from jax.experimental.pallas import tpu as pltpu
from jax.experimental.pallas import tpu_sc as plsc

V = 10000
E = 160000
B = 8
NCLS = 10

NC = 2
NS = 16
LANES = 16

VP = 10240
RPS = VP // NS
EPT = E // NS
NB = 80
NBAT = EPT // NB


def _make_spmm(D):
    NCH = max(D // 128, 1)
    RW = min(D, 128)
    Dc = RW // NC
    mesh = plsc.VectorSubcoreMesh(
        core_axis_name="c", subcore_axis_name="s", num_cores=NC, num_subcores=NS
    )

    NPAIR = (NBAT - 1) // 2
    RPT = V // NS

    @functools.partial(
        pl.kernel,
        out_type=jax.ShapeDtypeStruct((V, D), jnp.float32),
        mesh=mesh,
        scratch_types=[
            pltpu.VMEM((NBAT, NB), jnp.int32),
            pltpu.VMEM((NBAT, NB), jnp.int32),
            pltpu.VMEM((NBAT, NB), jnp.float32),
            pltpu.VMEM((NB,), jnp.int32),
            pltpu.VMEM((NB,), jnp.int32),
            pltpu.VMEM((NB, RW), jnp.float32),
            pltpu.VMEM((NB, RW), jnp.float32),
            pltpu.VMEM((NB, Dc), jnp.float32),
            pltpu.VMEM((NB, Dc), jnp.float32),
            pltpu.VMEM((NB, Dc), jnp.float32),
            pltpu.VMEM_SHARED((V, Dc), jnp.float32),
            pltpu.SemaphoreType.DMA,
            pltpu.SemaphoreType.DMA,
            pltpu.SemaphoreType.DMA,
            pltpu.SemaphoreType.DMA,
        ],
        compiler_params=pltpu.CompilerParams(use_tc_tiling_on_sc=False),
    )
    def spmm(x_hbm, src_hbm, dst_hbm, w_hbm, out_hbm,
             src2, dst2, w2, idxA, idxB, rA, rB, sbA, sbB, zbuf, acc,
             gsemA, gsemB, ssemA, ssemB):
        c = lax.axis_index("c")
        s = lax.axis_index("s")
        zeros = jnp.zeros((LANES,), jnp.float32)
        choff = c * Dc

        pltpu.sync_copy(src_hbm.at[s], src2)
        pltpu.sync_copy(dst_hbm.at[s], dst2)
        pltpu.sync_copy(w_hbm.at[s], w2)

        def zero_body(r, carry):
            for j in range(Dc // LANES):
                zbuf[r, pl.ds(j * LANES, LANES)] = zeros
            return carry
        lax.fori_loop(0, NB, zero_body, 0, unroll=4)

        def scale(i, rows, sbuf):
            def e_body(e, carry):
                g16 = (e // LANES) * LANES
                l = e - g16
                wg = w2[i, pl.ds(g16, LANES)]
                wv = wg.at[jnp.full((LANES,), l, jnp.int32)].get(
                    mode="promise_in_bounds")
                for j in range(Dc // LANES):
                    sbuf[e, pl.ds(j * LANES, LANES)] = (
                        rows[e, pl.ds(choff + j * LANES, LANES)] * wv)
                return carry
            lax.fori_loop(0, NB, e_body, 0, unroll=2)

        def drain_gather(rows, sem):
            pltpu.make_async_copy(x_hbm.at[pl.ds(0, NB)], rows, sem).wait()

        def start_scatter(i, sbuf, sem):
            pltpu.async_copy(sbuf, acc.at[dst2.at[i]], sem, add=True)

        def drain_scatter(sbuf, sem):
            pltpu.make_async_copy(
                x_hbm.at[pl.ds(0, NB), pl.ds(0, Dc)], sbuf, sem).wait()

        for t in range(TPC := NCH):
            def start_gather(i, idx_ref, rows, sem, _t=t):
                if NCH == 1:
                    pltpu.async_copy(x_hbm.at[src2.at[i]], rows, sem)
                else:
                    for g in range(NB // LANES):
                        sl = pl.ds(g * LANES, LANES)
                        idx_ref[sl] = src2[i, sl] * NCH + _t
                    pltpu.async_copy(x_hbm.at[idx_ref], rows, sem)

            for r in range(RPT // NB):
                pltpu.sync_copy(zbuf, acc.at[pl.ds(s * RPT + r * NB, NB)])
            pltpu.sync_copy(
                zbuf.at[pl.ds(0, RPT % NB)],
                acc.at[pl.ds(s * RPT + (RPT // NB) * NB, RPT % NB)],
            )
            plsc.subcore_barrier()

            start_gather(0, idxA, rA, gsemA)

            def pair_body(p, carry):
                i0 = 2 * p
                i1 = i0 + 1
                start_gather(i1, idxB, rB, gsemB)
                drain_gather(rA, gsemA)

                @pl.when(p > 0)
                def _():
                    drain_scatter(sbA, ssemA)
                scale(i0, rA, sbA)
                start_scatter(i0, sbA, ssemA)

                start_gather(i0 + 2, idxA, rA, gsemA)
                drain_gather(rB, gsemB)

                @pl.when(p > 0)
                def _():
                    drain_scatter(sbB, ssemB)
                scale(i1, rB, sbB)
                start_scatter(i1, sbB, ssemB)
                return carry
            lax.fori_loop(0, NPAIR, pair_body, 0)

            drain_gather(rA, gsemA)
            drain_scatter(sbA, ssemA)
            scale(NBAT - 1, rA, sbA)
            start_scatter(NBAT - 1, sbA, ssemA)
            drain_scatter(sbB, ssemB)
            drain_scatter(sbA, ssemA)
            plsc.subcore_barrier()

            pltpu.sync_copy(
                acc.at[pl.ds(s * RPT, RPT)],
                out_hbm.at[pl.ds(s * RPT, RPT), pl.ds(t * RW + choff, Dc)],
            )
            plsc.subcore_barrier()

    return spmm


_SPMM_KERNELS = {}


def _spmm(X, src, dst, w):
    D = X.shape[1]
    NCH = max(D // 128, 1)
    if D not in _SPMM_KERNELS:
        _SPMM_KERNELS[D] = _make_spmm(D)
    return _SPMM_KERNELS[D](X.reshape(V * NCH, min(D, 128)), src, dst, w)


def _cheb(h, src, dst, w, W, bb):
    C = h.shape[1]
    if C == 3:
        hp = jnp.concatenate(
            [h.reshape(V, B, C), jnp.zeros((V, B, 1), jnp.float32)], axis=2)
        x1 = _spmm(hp.reshape(V, B * 4), src, dst, w)
        x1 = x1.reshape(V, B, 4)[:, :, :3].reshape(V * B, C)
    else:
        x1 = _spmm(h.reshape(V, B * C), src, dst, w).reshape(V * B, C)
    return h @ W[0::2] + x1 @ W[1::2] + bb


def _bn_relu(h, g, b, eps=1e-5):
    mean = jnp.mean(h, axis=0)
    var = jnp.mean((h - mean) ** 2, axis=0)
    return jax.nn.relu((h - mean) / jnp.sqrt(var + eps) * g + b)


def _block(zin, p, src, dst, w):
    h = _bn_relu(zin, p["bn1_g"], p["bn1_b"])
    sc = (h @ p["sc_w"] + p["sc_b"]) if "sc_w" in p else zin
    h2 = _cheb(h, src, dst, w, p["w1"], p["b1"])
    h2 = _bn_relu(h2, p["bn2_g"], p["bn2_b"])
    h2 = _cheb(h2, src, dst, w, p["w2"], p["b2"])
    return sc + h2


def _head_body(z_ref, fcw_ref, fcb_ref, o_ref):
    m = jnp.max(z_ref[...], axis=0)
    logits = jnp.dot(m, fcw_ref[...], preferred_element_type=jnp.float32)
    logits = logits + fcb_ref[...][None, :]
    lse = jax.scipy.special.logsumexp(logits, axis=1, keepdims=True)
    o_ref[...] = logits - lse


def kernel(x, params, edge_index, edge_weight):
    src = edge_index[1].reshape(NS, NBAT, NB)
    dst = edge_index[0].reshape(NS, NBAT, NB)
    w = edge_weight.reshape(NS, NBAT, NB)

    z = jnp.transpose(x, (2, 0, 1)).reshape(V * B, 3)
    h = _cheb(z, src, dst, w, params["conv_w"], params["conv_b"])
    h = _block(h, params["block1"], src, dst, w)
    h = _block(h, params["block2"], src, dst, w)
    h = _block(h, params["block3"], src, dst, w)

    return pl.pallas_call(
        _head_body,
        out_shape=jax.ShapeDtypeStruct((B, NCLS), jnp.float32),
    )(h.reshape(V, B, 64), params["fc_w"], params["fc_b"])


# speedup vs baseline: 1.1225x; 1.1225x over previous
import functools

import jax
import jax.numpy as jnp
from jax import lax
from jax.experimental import pallas as pl
from jax.experimental.pallas import tpu as pltpu
from jax.experimental.pallas import tpu_sc as plsc

V = 10000
E = 160000
B = 8
NCLS = 10

NC = 2
NS = 16
LANES = 16

VP = 10240
RPS = VP // NS
EPT = E // NS
NB = 80
NBAT = EPT // NB


def _make_spmm(D):
    NCH = max(D // 128, 1)
    RW = min(D, 128)
    Dc = RW // NC
    mesh = plsc.VectorSubcoreMesh(
        core_axis_name="c", subcore_axis_name="s", num_cores=NC, num_subcores=NS
    )

    NPAIR = (NBAT - 1) // 2
    RPT = V // NS

    @functools.partial(
        pl.kernel,
        out_type=jax.ShapeDtypeStruct((V, D), jnp.float32),
        mesh=mesh,
        scratch_types=[
            pltpu.VMEM((NBAT, NB), jnp.int32),
            pltpu.VMEM((NBAT, NB), jnp.int32),
            pltpu.VMEM((NBAT, NB), jnp.float32),
            pltpu.VMEM((NB,), jnp.int32),
            pltpu.VMEM((NB,), jnp.int32),
            pltpu.VMEM((NB, RW), jnp.float32),
            pltpu.VMEM((NB, RW), jnp.float32),
            pltpu.VMEM((NB, Dc), jnp.float32),
            pltpu.VMEM((NB, Dc), jnp.float32),
            pltpu.VMEM((NB, Dc), jnp.float32),
            pltpu.VMEM_SHARED((V, Dc), jnp.float32),
            pltpu.SemaphoreType.DMA,
            pltpu.SemaphoreType.DMA,
            pltpu.SemaphoreType.DMA,
            pltpu.SemaphoreType.DMA,
        ],
        compiler_params=pltpu.CompilerParams(use_tc_tiling_on_sc=False),
    )
    def spmm(x_hbm, src_hbm, dst_hbm, w_hbm, out_hbm,
             src2, dst2, w2, idxA, idxB, rA, rB, sbA, sbB, zbuf, acc,
             gsemA, gsemB, ssemA, ssemB):
        c = lax.axis_index("c")
        s = lax.axis_index("s")
        zeros = jnp.zeros((LANES,), jnp.float32)
        choff = c * Dc

        pltpu.sync_copy(src_hbm.at[s], src2)
        pltpu.sync_copy(dst_hbm.at[s], dst2)
        pltpu.sync_copy(w_hbm.at[s], w2)

        def zero_body(r, carry):
            for j in range(Dc // LANES):
                zbuf[r, pl.ds(j * LANES, LANES)] = zeros
            return carry
        lax.fori_loop(0, NB, zero_body, 0, unroll=4)

        def scale(i, rows, sbuf):
            def e_body(e, carry):
                g16 = (e // LANES) * LANES
                l = e - g16
                wg = w2[i, pl.ds(g16, LANES)]
                wv = wg.at[jnp.full((LANES,), l, jnp.int32)].get(
                    mode="promise_in_bounds")
                for j in range(Dc // LANES):
                    sbuf[e, pl.ds(j * LANES, LANES)] = (
                        rows[e, pl.ds(choff + j * LANES, LANES)] * wv)
                return carry
            lax.fori_loop(0, NB, e_body, 0, unroll=2)

        def drain_gather(rows, sem):
            pltpu.make_async_copy(x_hbm.at[pl.ds(0, NB)], rows, sem).wait()

        def start_scatter(i, sbuf, sem):
            pltpu.async_copy(sbuf, acc.at[dst2.at[i]], sem, add=True)

        def drain_scatter(sbuf, sem):
            pltpu.make_async_copy(
                x_hbm.at[pl.ds(0, NB), pl.ds(0, Dc)], sbuf, sem).wait()

        for t in range(TPC := NCH):
            def start_gather(i, idx_ref, rows, sem, _t=t):
                if NCH == 1:
                    pltpu.async_copy(x_hbm.at[src2.at[i]], rows, sem)
                else:
                    for g in range(NB // LANES):
                        sl = pl.ds(g * LANES, LANES)
                        idx_ref[sl] = src2[i, sl] * NCH + _t
                    pltpu.async_copy(x_hbm.at[idx_ref], rows, sem)

            for r in range(RPT // NB):
                pltpu.sync_copy(zbuf, acc.at[pl.ds(s * RPT + r * NB, NB)])
            pltpu.sync_copy(
                zbuf.at[pl.ds(0, RPT % NB)],
                acc.at[pl.ds(s * RPT + (RPT // NB) * NB, RPT % NB)],
            )
            plsc.subcore_barrier()

            start_gather(0, idxA, rA, gsemA)

            def pair_body(p, carry):
                i0 = 2 * p
                i1 = i0 + 1
                start_gather(i1, idxB, rB, gsemB)
                drain_gather(rA, gsemA)

                @pl.when(p > 0)
                def _():
                    drain_scatter(sbA, ssemA)
                scale(i0, rA, sbA)
                start_scatter(i0, sbA, ssemA)

                start_gather(i0 + 2, idxA, rA, gsemA)
                drain_gather(rB, gsemB)

                @pl.when(p > 0)
                def _():
                    drain_scatter(sbB, ssemB)
                scale(i1, rB, sbB)
                start_scatter(i1, sbB, ssemB)
                return carry
            lax.fori_loop(0, NPAIR, pair_body, 0)

            drain_gather(rA, gsemA)
            drain_scatter(sbA, ssemA)
            scale(NBAT - 1, rA, sbA)
            start_scatter(NBAT - 1, sbA, ssemA)
            drain_scatter(sbB, ssemB)
            drain_scatter(sbA, ssemA)
            plsc.subcore_barrier()

            pltpu.sync_copy(
                acc.at[pl.ds(s * RPT, RPT)],
                out_hbm.at[pl.ds(s * RPT, RPT), pl.ds(t * RW + choff, Dc)],
            )
            plsc.subcore_barrier()

    return spmm


_SPMM_KERNELS = {}


def _spmm(X, src, dst, w):
    D = X.shape[1]
    NCH = max(D // 128, 1)
    if D not in _SPMM_KERNELS:
        _SPMM_KERNELS[D] = _make_spmm(D)
    return _SPMM_KERNELS[D](X.reshape(V * NCH, min(D, 128)), src, dst, w)


VB = 2000
NBLK = V // VB


def _conv_call(x0, s1, w0b, w1b, bv, resid=None, scwb=None, scbv=None):
    Din = x0.shape[1]
    Dout = w0b.shape[1]
    has_r = resid is not None
    has_sc = scwb is not None
    Dsc = scwb.shape[1] if has_sc else 0

    def body(*refs):
        i = 0
        x0_ref = refs[i]; i += 1
        s1_ref = refs[i]; i += 1
        r_ref = None
        if has_r:
            r_ref = refs[i]; i += 1
        w0_ref = refs[i]; i += 1
        w1_ref = refs[i]; i += 1
        b_ref = refs[i]; i += 1
        scw_ref = scb_ref = None
        if has_sc:
            scw_ref = refs[i]; i += 1
            scb_ref = refs[i]; i += 1
        h_ref = refs[i]; i += 1
        p_ref = refs[i]; i += 1
        sc_ref = refs[i] if has_sc else None

        h = jnp.dot(x0_ref[...], w0_ref[...],
                    preferred_element_type=jnp.float32)
        h = h + jnp.dot(s1_ref[...], w1_ref[...],
                        preferred_element_type=jnp.float32)
        h = h + b_ref[...]
        if has_r:
            h = h + r_ref[...]
        h_ref[...] = h
        p_ref[0, 0, :] = jnp.sum(h, axis=0)
        p_ref[0, 1, :] = jnp.sum(h * h, axis=0)
        if has_sc:
            sc_ref[...] = (jnp.dot(x0_ref[...], scw_ref[...],
                                   preferred_element_type=jnp.float32)
                           + scb_ref[...])

    full = lambda shp: pl.BlockSpec(shp, lambda i: (0,) * len(shp))
    rows = lambda d: pl.BlockSpec((VB, d), lambda i: (i, 0))
    in_specs = [rows(Din), rows(Din)]
    args = [x0, s1]
    if has_r:
        in_specs.append(rows(Dout))
        args.append(resid)
    in_specs += [full((Din, Dout)), full((Din, Dout)), full((1, Dout))]
    args += [w0b, w1b, bv]
    if has_sc:
        in_specs += [full((Din, Dsc)), full((1, Dsc))]
        args += [scwb, scbv]
    out_shape = [jax.ShapeDtypeStruct((V, Dout), jnp.float32),
                 jax.ShapeDtypeStruct((NBLK, 2, Dout), jnp.float32)]
    out_specs = [rows(Dout), pl.BlockSpec((1, 2, Dout), lambda i: (i, 0, 0))]
    if has_sc:
        out_shape.append(jax.ShapeDtypeStruct((V, Dsc), jnp.float32))
        out_specs.append(rows(Dsc))
    res = pl.pallas_call(
        body, grid=(NBLK,), in_specs=in_specs, out_specs=out_specs,
        out_shape=out_shape,
    )(*args)
    return res


def _bn_relu_call(h, p, kmat, ktmat, g2, b2, eps=1e-5):
    D = h.shape[1]
    Cout = kmat.shape[1]
    n = float(V * B)

    def body(h_ref, p_ref, k_ref, kt_ref, g_ref, b_ref, a_ref):
        s1 = jnp.sum(p_ref[:, 0, :], axis=0)[None, :]
        s2 = jnp.sum(p_ref[:, 1, :], axis=0)[None, :]
        m1 = jnp.dot(s1, k_ref[...], preferred_element_type=jnp.float32) / n
        m2 = jnp.dot(s2, k_ref[...], preferred_element_type=jnp.float32) / n
        var = m2 - m1 * m1
        scale = g_ref[...] * lax.rsqrt(var + eps)
        shift = b_ref[...] - m1 * scale
        s_cols = jnp.dot(scale, kt_ref[...], preferred_element_type=jnp.float32)
        t_cols = jnp.dot(shift, kt_ref[...], preferred_element_type=jnp.float32)
        a_ref[...] = jnp.maximum(h_ref[...] * s_cols + t_cols, 0.0)

    full = lambda shp: pl.BlockSpec(shp, lambda i: (0,) * len(shp))
    return pl.pallas_call(
        body, grid=(NBLK,),
        in_specs=[pl.BlockSpec((VB, D), lambda i: (i, 0)),
                  full((NBLK, 2, D)), full((D, Cout)), full((Cout, D)),
                  full((1, Cout)), full((1, Cout))],
        out_specs=pl.BlockSpec((VB, D), lambda i: (i, 0)),
        out_shape=jax.ShapeDtypeStruct((V, D), jnp.float32),
    )(h, p, kmat, ktmat, g2, b2)


def _head_body(z_ref, fcw_ref, fcb_ref, o_ref):
    m = jnp.max(z_ref[...], axis=0)
    logits = jnp.dot(m, fcw_ref[...], preferred_element_type=jnp.float32)
    logits = logits + fcb_ref[...][None, :]
    lse = jax.scipy.special.logsumexp(logits, axis=1, keepdims=True)
    o_ref[...] = logits - lse


def _kron_b(w):
    return jnp.kron(jnp.eye(B, dtype=jnp.float32), w)


def _bn_mats(cout):
    k = jnp.tile(jnp.eye(cout, dtype=jnp.float32), (B, 1))
    return k, k.T


def kernel(x, params, edge_index, edge_weight):
    src = edge_index[1].reshape(NS, NBAT, NB)
    dst = edge_index[0].reshape(NS, NBAT, NB)
    w = edge_weight.reshape(NS, NBAT, NB)
    p = params

    z0 = jnp.transpose(x, (2, 0, 1))
    z0 = jnp.concatenate(
        [z0, jnp.zeros((V, B, 1), jnp.float32)], axis=2).reshape(V, B * 4)

    def wpair(wmat, cin_pad=None):
        w0, w1 = wmat[0::2], wmat[1::2]
        if cin_pad is not None:
            zpad = jnp.zeros((cin_pad - w0.shape[0], w0.shape[1]), jnp.float32)
            w0 = jnp.concatenate([w0, zpad], axis=0)
            w1 = jnp.concatenate([w1, zpad], axis=0)
        return _kron_b(w0), _kron_b(w1)

    def btile(b):
        return jnp.tile(b, B)[None, :]

    s1 = _spmm(z0, src, dst, w)
    w0b, w1b = wpair(p["conv_w"], cin_pad=4)
    h1, p1 = _conv_call(z0, s1, w0b, w1b, btile(p["conv_b"]))

    b1 = p["block1"]
    k16, kt16 = _bn_mats(16)
    a1 = _bn_relu_call(h1, p1, k16, kt16, b1["bn1_g"][None], b1["bn1_b"][None])
    s2 = _spmm(a1, src, dst, w)
    w0b, w1b = wpair(b1["w1"])
    h2, p2 = _conv_call(a1, s2, w0b, w1b, btile(b1["b1"]))
    a2 = _bn_relu_call(h2, p2, k16, kt16, b1["bn2_g"][None], b1["bn2_b"][None])
    s3 = _spmm(a2, src, dst, w)
    w0b, w1b = wpair(b1["w2"])
    z1, p3 = _conv_call(a2, s3, w0b, w1b, btile(b1["b2"]), resid=h1)

    b2 = p["block2"]
    k32, kt32 = _bn_mats(32)
    a3 = _bn_relu_call(z1, p3, k16, kt16, b2["bn1_g"][None], b2["bn1_b"][None])
    s4 = _spmm(a3, src, dst, w)
    w0b, w1b = wpair(b2["w1"])
    h4, p4, sc2 = _conv_call(a3, s4, w0b, w1b, btile(b2["b1"]),
                             scwb=_kron_b(b2["sc_w"]), scbv=btile(b2["sc_b"]))
    a4 = _bn_relu_call(h4, p4, k32, kt32, b2["bn2_g"][None], b2["bn2_b"][None])
    s5 = _spmm(a4, src, dst, w)
    w0b, w1b = wpair(b2["w2"])
    z2, p5 = _conv_call(a4, s5, w0b, w1b, btile(b2["b2"]), resid=sc2)

    b3 = p["block3"]
    k64, kt64 = _bn_mats(64)
    a5 = _bn_relu_call(z2, p5, k32, kt32, b3["bn1_g"][None], b3["bn1_b"][None])
    s6 = _spmm(a5, src, dst, w)
    w0b, w1b = wpair(b3["w1"])
    h6, p6, sc3 = _conv_call(a5, s6, w0b, w1b, btile(b3["b1"]),
                             scwb=_kron_b(b3["sc_w"]), scbv=btile(b3["sc_b"]))
    a6 = _bn_relu_call(h6, p6, k64, kt64, b3["bn2_g"][None], b3["bn2_b"][None])
    s7 = _spmm(a6, src, dst, w)
    w0b, w1b = wpair(b3["w2"])
    z3, _ = _conv_call(a6, s7, w0b, w1b, btile(b3["b2"]), resid=sc3)

    return pl.pallas_call(
        _head_body,
        out_shape=jax.ShapeDtypeStruct((B, NCLS), jnp.float32),
    )(z3.reshape(V, B, 64), params["fc_w"], params["fc_b"])
